# Initial kernel scaffold; baseline (speedup 1.0000x reference)
import jax
import jax.numpy as jnp
from jax.experimental import pallas as pl


def kernel(x0, edge_index0, edge_attr0, batch0, x1, edge_index1, edge_attr1, batch1, enc0, enc1, proj):
    raise NotImplementedError("write your pallas kernel here")


# speedup vs baseline: 2.9072x; 2.9072x over previous
import functools

import jax
import jax.numpy as jnp
from jax import lax
from jax.experimental import pallas as pl
from jax.experimental.pallas import tpu as pltpu
from jax.experimental.pallas import tpu_sc as plsc

_EMB = 300
_PAD = 320
_HALF = 160
_HID = 640
_N = 10000
_E = 160000
_G = 512
_LAYERS = 5
_TEMP = 0.04
_TILES = 16
_CHUNK = 125
_EPT = _E // _TILES
_NCH = _EPT // _CHUNK
_ROWS_PT = 624
_RB = 1000
_GRID = _N // _RB
_PREC = jax.lax.Precision.HIGHEST


@functools.lru_cache(maxsize=None)
def _make_sc_scatter(d_half):
  mesh = plsc.VectorSubcoreMesh(core_axis_name="c", subcore_axis_name="s")
  out_t = (jax.ShapeDtypeStruct((_N, d_half), jnp.float32),
           jax.ShapeDtypeStruct((_N, d_half), jnp.float32))

  @functools.partial(
      pl.kernel,
      out_type=out_t,
      mesh=mesh,
      compiler_params=pltpu.CompilerParams(use_tc_tiling_on_sc=False),
      scratch_types=[
          pltpu.VMEM((_CHUNK,), jnp.int32),
          pltpu.VMEM((_CHUNK,), jnp.int32),
          pltpu.VMEM((_CHUNK, d_half), jnp.float32),
          pltpu.VMEM_SHARED((_N, d_half), jnp.float32),
          pltpu.SemaphoreType.DMA,
      ],
  )
  def sc_kernel(tab_a, tab_b, idx4, seg4, zeros_hbm, out_a, out_b,
                idx_v, seg_v, rows_v, acc_sh, sem):
    c = lax.axis_index("c")
    s = lax.axis_index("s")

    def tile_rows_copy(src, dst):
      pltpu.sync_copy(src.at[pl.ds(s * _ROWS_PT, _ROWS_PT)],
                      dst.at[pl.ds(s * _ROWS_PT, _ROWS_PT)])

      @pl.when(s == _TILES - 1)
      def _():
        pltpu.sync_copy(src.at[pl.ds(_TILES * _ROWS_PT, _N - _TILES * _ROWS_PT)],
                        dst.at[pl.ds(_TILES * _ROWS_PT, _N - _TILES * _ROWS_PT)])

    tile_rows_copy(zeros_hbm, acc_sh)
    plsc.subcore_barrier()

    def run(tab):
      def step(i, carry):
        pltpu.sync_copy(idx4.at[i, s, 0], idx_v)
        pltpu.sync_copy(seg4.at[i, s, 0], seg_v)
        pltpu.async_copy(tab.at[idx_v], rows_v, sem).wait()
        pltpu.sync_copy(rows_v, acc_sh.at[seg_v], add=True)
        return carry
      lax.fori_loop(0, _NCH, step, 0)

    @pl.when(c == 0)
    def _():
      run(tab_a)

    @pl.when(c == 1)
    def _():
      run(tab_b)

    plsc.subcore_barrier()

    @pl.when(c == 0)
    def _():
      tile_rows_copy(acc_sh, out_a)

    @pl.when(c == 1)
    def _():
      tile_rows_copy(acc_sh, out_b)

  return sc_kernel


def _sc_scatter_h(*a):
  return _make_sc_scatter(_HALF)(*a)


def _sc_scatter_cnt(*a):
  return _make_sc_scatter(16)(*a)


def _embed_body(cx_ref, tab_ref, out_ref):
  c = cx_ref[...]
  oh = (lax.broadcasted_iota(jnp.int32, (_RB, 16), 1).astype(jnp.float32) == c)
  out_ref[...] = jnp.dot(oh.astype(jnp.float32), tab_ref[...],
                         preferred_element_type=jnp.float32, precision=_PREC)


def _embed(cx, tab):
  return pl.pallas_call(
      _embed_body,
      grid=(_GRID,),
      in_specs=[
          pl.BlockSpec((_RB, 1), lambda i: (i, 0)),
          pl.BlockSpec((16, _PAD), lambda i: (0, 0)),
      ],
      out_specs=pl.BlockSpec((_RB, _PAD), lambda i: (i, 0)),
      out_shape=jax.ShapeDtypeStruct((_N, _PAD), jnp.float32),
  )(cx, tab)


def _mlp_body(agg_ref, h_ref, c_ref, combo_ref, selfrow_ref,
              w1_ref, b1_ref, w2_ref, b2_ref, y_ref):
  pre = (agg_ref[...] + h_ref[...] + selfrow_ref[...]
         + jnp.dot(c_ref[...], combo_ref[...],
                   preferred_element_type=jnp.float32, precision=_PREC))
  t = jnp.maximum(
      jnp.dot(pre, w1_ref[...], preferred_element_type=jnp.float32,
              precision=_PREC) + b1_ref[...], 0.0)
  y_ref[...] = jnp.dot(t, w2_ref[...], preferred_element_type=jnp.float32,
                       precision=_PREC) + b2_ref[...]


def _mlp(agg, h, cmat, combo, selfrow, w1, b1, w2, b2):
  return pl.pallas_call(
      _mlp_body,
      grid=(_GRID,),
      in_specs=[
          pl.BlockSpec((_RB, _PAD), lambda i: (i, 0)),
          pl.BlockSpec((_RB, _PAD), lambda i: (i, 0)),
          pl.BlockSpec((_RB, 16), lambda i: (i, 0)),
          pl.BlockSpec((16, _PAD), lambda i: (0, 0)),
          pl.BlockSpec((1, _PAD), lambda i: (0, 0)),
          pl.BlockSpec((_PAD, _HID), lambda i: (0, 0)),
          pl.BlockSpec((1, _HID), lambda i: (0, 0)),
          pl.BlockSpec((_HID, _PAD), lambda i: (0, 0)),
          pl.BlockSpec((1, _PAD), lambda i: (0, 0)),
      ],
      out_specs=pl.BlockSpec((_RB, _PAD), lambda i: (i, 0)),
      out_shape=jax.ShapeDtypeStruct((_N, _PAD), jnp.float32),
  )(agg, h, cmat, combo, selfrow, w1, b1, w2, b2)


def _make_bn(do_relu):
  def body(y_ref, g_ref, b_ref, out_ref):
    y = y_ref[...]
    mu = jnp.mean(y, axis=0, keepdims=True)
    d = y - mu
    var = jnp.mean(d * d, axis=0, keepdims=True)
    h = d / jnp.sqrt(var + 1e-5) * g_ref[...] + b_ref[...]
    if do_relu:
      h = jnp.maximum(h, 0.0)
    out_ref[...] = h

  def bn(y, g, b):
    return pl.pallas_call(
        body,
        out_shape=jax.ShapeDtypeStruct((_N, _PAD), jnp.float32),
    )(y, g, b)
  return bn


_bn_relu = _make_bn(True)
_bn_last = _make_bn(False)


def _pool_body(b_ref, h_ref, psum_ref, pcnt_ref):
  @pl.when(pl.program_id(0) == 0)
  def _():
    psum_ref[...] = jnp.zeros_like(psum_ref)
    pcnt_ref[...] = jnp.zeros_like(pcnt_ref)
  brow = b_ref[...].reshape(1, _RB)
  oh_t = (lax.broadcasted_iota(jnp.int32, (_G, _RB), 0).astype(jnp.float32)
          == brow)
  oh_t = oh_t.astype(jnp.float32)
  psum_ref[...] += jnp.dot(oh_t, h_ref[...],
                           preferred_element_type=jnp.float32,
                           precision=_PREC)
  cnt = jnp.sum(oh_t, axis=1, keepdims=True)
  pcnt_ref[...] += jnp.broadcast_to(cnt, (_G, 128))


def _pool(batch3d, h):
  return pl.pallas_call(
      _pool_body,
      grid=(_GRID,),
      in_specs=[
          pl.BlockSpec((1, 1, _RB), lambda i: (i, 0, 0)),
          pl.BlockSpec((_RB, _PAD), lambda i: (i, 0)),
      ],
      out_specs=[
          pl.BlockSpec((_G, _PAD), lambda i: (0, 0)),
          pl.BlockSpec((_G, 128), lambda i: (0, 0)),
      ],
      out_shape=[
          jax.ShapeDtypeStruct((_G, _PAD), jnp.float32),
          jax.ShapeDtypeStruct((_G, 128), jnp.float32),
      ],
  )(batch3d, h)


def _head_body(p0_ref, c0_ref, p1_ref, c1_ref, w1_ref, b1_ref, w2_ref,
               b2_ref, out_ref):
  def feat(p_ref, c_ref):
    cnt = jnp.maximum(c_ref[:, 0:1], 1.0)
    m = p_ref[...] / cnt
    t = jnp.maximum(
        jnp.dot(m, w1_ref[...], preferred_element_type=jnp.float32,
                precision=_PREC) + b1_ref[...], 0.0)
    z = jnp.dot(t, w2_ref[...], preferred_element_type=jnp.float32,
                precision=_PREC) + b2_ref[...]
    nrm = jnp.sqrt(jnp.sum(z * z, axis=1, keepdims=True))
    return z / jnp.maximum(nrm, 1e-12)
  f0 = feat(p0_ref, c0_ref)
  f1 = feat(p1_ref, c1_ref)
  out_ref[...] = lax.dot_general(
      f0, f1, (((1,), (1,)), ((), ())),
      preferred_element_type=jnp.float32, precision=_PREC) * (1.0 / _TEMP)


def _head(p0, c0, p1, c1, w1, b1, w2, b2):
  return pl.pallas_call(
      _head_body,
      out_shape=jax.ShapeDtypeStruct((_G, _G), jnp.float32),
  )(p0, c0, p1, c1, w1, b1, w2, b2)


def _pad_to(a, shape):
  pads = [(0, t - s) for s, t in zip(a.shape, shape)]
  return jnp.pad(a, pads)


def _prep_encoder(p):
  a = jnp.arange(9)
  t9 = p["atom_emb1"][a // 3] + p["atom_emb2"][a % 3]
  t9 = _pad_to(t9, (16, _PAD))
  combo = p["ee1"][:, a // 3, :] + p["ee2"][:, a % 3, :]
  combo = _pad_to(combo, (_LAYERS, 16, _PAD))
  selfrow = (p["ee1"][:, 4, :] + p["ee2"][:, 0, :])[:, None, :]
  selfrow = _pad_to(selfrow, (_LAYERS, 1, _PAD))
  w1 = _pad_to(p["W1"], (_LAYERS, _PAD, _HID))
  b1 = _pad_to(p["b1"], (_LAYERS, _HID))[:, None, :]
  w2 = _pad_to(p["W2"], (_LAYERS, _HID, _PAD))
  b2 = _pad_to(p["b2"], (_LAYERS, _PAD))[:, None, :]
  g = _pad_to(p["bn_g"], (_LAYERS, _PAD))[:, None, :]
  b = _pad_to(p["bn_b"], (_LAYERS, _PAD))[:, None, :]
  return t9, combo, selfrow, w1, b1, w2, b2, g, b


def _encode(x, edge_index, edge_attr, batch, p, zeros_n, zeros_c, eye9):
  t9, combo, selfrow, w1, b1, w2, b2, g, b = _prep_encoder(p)
  cx = (x[:, 0] * 3 + x[:, 1]).astype(jnp.float32)[:, None]

  def to4d(v):
    return jnp.transpose(v.reshape(_TILES, _NCH, 1, _CHUNK), (1, 0, 2, 3))

  src2 = to4d(edge_index[0])
  dst2 = to4d(edge_index[1])
  ce2 = to4d((edge_attr[:, 0] * 3 + edge_attr[:, 1]).astype(jnp.int32))
  batch3d = batch.astype(jnp.float32).reshape(_GRID, 1, _RB)

  cmat, _ = _sc_scatter_cnt(eye9, eye9, ce2, dst2, zeros_c)

  h = _embed(cx, t9)
  for l in range(_LAYERS):
    h_a = h[:, :_HALF]
    h_b = h[:, _HALF:]
    agg_a, agg_b = _sc_scatter_h(h_a, h_b, src2, dst2, zeros_n)
    agg = jnp.concatenate([agg_a, agg_b], axis=1)
    y = _mlp(agg, h, cmat, combo[l], selfrow[l], w1[l], b1[l], w2[l], b2[l])
    h = (_bn_relu if l < _LAYERS - 1 else _bn_last)(y, g[l], b[l])
  return _pool(batch3d, h)


def kernel(x0, edge_index0, edge_attr0, batch0, x1, edge_index1, edge_attr1,
           batch1, enc0, enc1, proj):
  zeros_n = jnp.zeros((_N, _HALF), jnp.float32)
  zeros_c = jnp.zeros((_N, 16), jnp.float32)
  eye9 = jnp.eye(9, 16, dtype=jnp.float32)

  p0, c0 = _encode(x0, edge_index0, edge_attr0, batch0, enc0,
                   zeros_n, zeros_c, eye9)
  p1, c1 = _encode(x1, edge_index1, edge_attr1, batch1, enc1,
                   zeros_n, zeros_c, eye9)

  pw1 = _pad_to(proj["W1"], (_PAD, _PAD))
  pb1 = _pad_to(proj["b1"], (_PAD,))[None, :]
  pw2 = _pad_to(proj["W2"], (_PAD, _PAD))
  pb2 = _pad_to(proj["b2"], (_PAD,))[None, :]
  logits = _head(p0, c0, p1, c1, pw1, pb1, pw2, pb2)
  labels = jnp.arange(_G, dtype=jnp.int32)
  return (logits, labels)


# speedup vs baseline: 3.2251x; 1.1094x over previous
import functools

import jax
import jax.numpy as jnp
from jax import lax
from jax.experimental import pallas as pl
from jax.experimental.pallas import tpu as pltpu
from jax.experimental.pallas import tpu_sc as plsc

_EMB = 300
_PAD = 320
_HALF = 160
_HID = 640
_N = 10000
_E = 160000
_G = 512
_LAYERS = 5
_TEMP = 0.04
_TILES = 16
_CHUNK = 80
_EPT = _E // _TILES
_NCH = _EPT // _CHUNK
_ROWS_PT = 624
_RB = 1000
_GRID = _N // _RB
_PREC = jax.lax.Precision.HIGHEST


@functools.lru_cache(maxsize=None)
def _make_sc_scatter(d_half):
  mesh = plsc.VectorSubcoreMesh(core_axis_name="c", subcore_axis_name="s")
  out_t = (jax.ShapeDtypeStruct((_N, d_half), jnp.float32),
           jax.ShapeDtypeStruct((_N, d_half), jnp.float32))

  @functools.partial(
      pl.kernel,
      out_type=out_t,
      mesh=mesh,
      compiler_params=pltpu.CompilerParams(use_tc_tiling_on_sc=False),
      scratch_types=[
          pltpu.VMEM((_CHUNK,), jnp.int32),
          pltpu.VMEM((_CHUNK,), jnp.int32),
          pltpu.VMEM((_CHUNK,), jnp.int32),
          pltpu.VMEM((_CHUNK,), jnp.int32),
          pltpu.VMEM((_CHUNK, d_half), jnp.float32),
          pltpu.VMEM((_CHUNK, d_half), jnp.float32),
          pltpu.VMEM_SHARED((_N, d_half), jnp.float32),
          pltpu.SemaphoreType.DMA,
          pltpu.SemaphoreType.DMA,
      ],
  )
  def sc_kernel(tab_a, tab_b, idx4, seg4, zeros_hbm, out_a, out_b,
                idx_p, seg_p, idx_q, seg_q, rows_p, rows_q, acc_sh,
                gsem_p, gsem_q):
    c = lax.axis_index("c")
    s = lax.axis_index("s")

    def tile_rows_copy(src, dst):
      pltpu.sync_copy(src.at[pl.ds(s * _ROWS_PT, _ROWS_PT)],
                      dst.at[pl.ds(s * _ROWS_PT, _ROWS_PT)])

      @pl.when(s == _TILES - 1)
      def _():
        pltpu.sync_copy(src.at[pl.ds(_TILES * _ROWS_PT, _N - _TILES * _ROWS_PT)],
                        dst.at[pl.ds(_TILES * _ROWS_PT, _N - _TILES * _ROWS_PT)])

    tile_rows_copy(zeros_hbm, acc_sh)
    plsc.subcore_barrier()

    def run(tab):
      pltpu.sync_copy(idx4.at[0, s, 0], idx_p)
      pltpu.sync_copy(seg4.at[0, s, 0], seg_p)
      pltpu.async_copy(tab.at[idx_p], rows_p, gsem_p)

      @pl.loop(0, _NCH + 1, step=2)
      def _(i):
        @pl.when(i + 1 < _NCH)
        def _():
          pltpu.sync_copy(idx4.at[i + 1, s, 0], idx_q)
          pltpu.sync_copy(seg4.at[i + 1, s, 0], seg_q)
          pltpu.async_copy(tab.at[idx_q], rows_q, gsem_q)
        pltpu.make_async_copy(tab.at[idx_p], rows_p, gsem_p).wait()
        pltpu.sync_copy(rows_p, acc_sh.at[seg_p], add=True)

        @pl.when(i + 2 < _NCH)
        def _():
          pltpu.sync_copy(idx4.at[i + 2, s, 0], idx_p)
          pltpu.sync_copy(seg4.at[i + 2, s, 0], seg_p)
          pltpu.async_copy(tab.at[idx_p], rows_p, gsem_p)

        @pl.when(i + 1 < _NCH)
        def _():
          pltpu.make_async_copy(tab.at[idx_q], rows_q, gsem_q).wait()
          pltpu.sync_copy(rows_q, acc_sh.at[seg_q], add=True)

    @pl.when(c == 0)
    def _():
      run(tab_a)

    @pl.when(c == 1)
    def _():
      run(tab_b)

    plsc.subcore_barrier()

    @pl.when(c == 0)
    def _():
      tile_rows_copy(acc_sh, out_a)

    @pl.when(c == 1)
    def _():
      tile_rows_copy(acc_sh, out_b)

  return sc_kernel


def _sc_scatter_h(*a):
  return _make_sc_scatter(_HALF)(*a)


def _sc_scatter_cnt(*a):
  return _make_sc_scatter(16)(*a)


def _embed_body(cx_ref, tab_ref, out_ref):
  c = cx_ref[...]
  oh = (lax.broadcasted_iota(jnp.int32, (_RB, 16), 1).astype(jnp.float32) == c)
  out_ref[...] = jnp.dot(oh.astype(jnp.float32), tab_ref[...],
                         preferred_element_type=jnp.float32, precision=_PREC)


def _embed(cx, tab):
  return pl.pallas_call(
      _embed_body,
      grid=(_GRID,),
      in_specs=[
          pl.BlockSpec((_RB, 1), lambda i: (i, 0)),
          pl.BlockSpec((16, _PAD), lambda i: (0, 0)),
      ],
      out_specs=pl.BlockSpec((_RB, _PAD), lambda i: (i, 0)),
      out_shape=jax.ShapeDtypeStruct((_N, _PAD), jnp.float32),
  )(cx, tab)


def _mlp_body(agg_ref, h_ref, c_ref, combo_ref, selfrow_ref,
              w1_ref, b1_ref, w2_ref, b2_ref, y_ref):
  pre = (agg_ref[...] + h_ref[...] + selfrow_ref[...]
         + jnp.dot(c_ref[...], combo_ref[...],
                   preferred_element_type=jnp.float32, precision=_PREC))
  t = jnp.maximum(
      jnp.dot(pre, w1_ref[...], preferred_element_type=jnp.float32,
              precision=_PREC) + b1_ref[...], 0.0)
  y_ref[...] = jnp.dot(t, w2_ref[...], preferred_element_type=jnp.float32,
                       precision=_PREC) + b2_ref[...]


def _mlp(agg, h, cmat, combo, selfrow, w1, b1, w2, b2):
  return pl.pallas_call(
      _mlp_body,
      grid=(_GRID,),
      in_specs=[
          pl.BlockSpec((_RB, _PAD), lambda i: (i, 0)),
          pl.BlockSpec((_RB, _PAD), lambda i: (i, 0)),
          pl.BlockSpec((_RB, 16), lambda i: (i, 0)),
          pl.BlockSpec((16, _PAD), lambda i: (0, 0)),
          pl.BlockSpec((1, _PAD), lambda i: (0, 0)),
          pl.BlockSpec((_PAD, _HID), lambda i: (0, 0)),
          pl.BlockSpec((1, _HID), lambda i: (0, 0)),
          pl.BlockSpec((_HID, _PAD), lambda i: (0, 0)),
          pl.BlockSpec((1, _PAD), lambda i: (0, 0)),
      ],
      out_specs=pl.BlockSpec((_RB, _PAD), lambda i: (i, 0)),
      out_shape=jax.ShapeDtypeStruct((_N, _PAD), jnp.float32),
  )(agg, h, cmat, combo, selfrow, w1, b1, w2, b2)


def _make_bn(do_relu):
  def body(y_ref, g_ref, b_ref, out_ref):
    y = y_ref[...]
    mu = jnp.mean(y, axis=0, keepdims=True)
    d = y - mu
    var = jnp.mean(d * d, axis=0, keepdims=True)
    h = d / jnp.sqrt(var + 1e-5) * g_ref[...] + b_ref[...]
    if do_relu:
      h = jnp.maximum(h, 0.0)
    out_ref[...] = h

  def bn(y, g, b):
    return pl.pallas_call(
        body,
        out_shape=jax.ShapeDtypeStruct((_N, _PAD), jnp.float32),
    )(y, g, b)
  return bn


_bn_relu = _make_bn(True)
_bn_last = _make_bn(False)


def _pool_body(b_ref, h_ref, psum_ref, pcnt_ref):
  @pl.when(pl.program_id(0) == 0)
  def _():
    psum_ref[...] = jnp.zeros_like(psum_ref)
    pcnt_ref[...] = jnp.zeros_like(pcnt_ref)
  brow = b_ref[...].reshape(1, _RB)
  oh_t = (lax.broadcasted_iota(jnp.int32, (_G, _RB), 0).astype(jnp.float32)
          == brow)
  oh_t = oh_t.astype(jnp.float32)
  psum_ref[...] += jnp.dot(oh_t, h_ref[...],
                           preferred_element_type=jnp.float32,
                           precision=_PREC)
  cnt = jnp.sum(oh_t, axis=1, keepdims=True)
  pcnt_ref[...] += jnp.broadcast_to(cnt, (_G, 128))


def _pool(batch3d, h):
  return pl.pallas_call(
      _pool_body,
      grid=(_GRID,),
      in_specs=[
          pl.BlockSpec((1, 1, _RB), lambda i: (i, 0, 0)),
          pl.BlockSpec((_RB, _PAD), lambda i: (i, 0)),
      ],
      out_specs=[
          pl.BlockSpec((_G, _PAD), lambda i: (0, 0)),
          pl.BlockSpec((_G, 128), lambda i: (0, 0)),
      ],
      out_shape=[
          jax.ShapeDtypeStruct((_G, _PAD), jnp.float32),
          jax.ShapeDtypeStruct((_G, 128), jnp.float32),
      ],
  )(batch3d, h)


def _head_body(p0_ref, c0_ref, p1_ref, c1_ref, w1_ref, b1_ref, w2_ref,
               b2_ref, out_ref):
  def feat(p_ref, c_ref):
    cnt = jnp.maximum(c_ref[:, 0:1], 1.0)
    m = p_ref[...] / cnt
    t = jnp.maximum(
        jnp.dot(m, w1_ref[...], preferred_element_type=jnp.float32,
                precision=_PREC) + b1_ref[...], 0.0)
    z = jnp.dot(t, w2_ref[...], preferred_element_type=jnp.float32,
                precision=_PREC) + b2_ref[...]
    nrm = jnp.sqrt(jnp.sum(z * z, axis=1, keepdims=True))
    return z / jnp.maximum(nrm, 1e-12)
  f0 = feat(p0_ref, c0_ref)
  f1 = feat(p1_ref, c1_ref)
  out_ref[...] = lax.dot_general(
      f0, f1, (((1,), (1,)), ((), ())),
      preferred_element_type=jnp.float32, precision=_PREC) * (1.0 / _TEMP)


def _head(p0, c0, p1, c1, w1, b1, w2, b2):
  return pl.pallas_call(
      _head_body,
      out_shape=jax.ShapeDtypeStruct((_G, _G), jnp.float32),
  )(p0, c0, p1, c1, w1, b1, w2, b2)


def _pad_to(a, shape):
  pads = [(0, t - s) for s, t in zip(a.shape, shape)]
  return jnp.pad(a, pads)


def _prep_encoder(p):
  a = jnp.arange(9)
  t9 = p["atom_emb1"][a // 3] + p["atom_emb2"][a % 3]
  t9 = _pad_to(t9, (16, _PAD))
  combo = p["ee1"][:, a // 3, :] + p["ee2"][:, a % 3, :]
  combo = _pad_to(combo, (_LAYERS, 16, _PAD))
  selfrow = (p["ee1"][:, 4, :] + p["ee2"][:, 0, :])[:, None, :]
  selfrow = _pad_to(selfrow, (_LAYERS, 1, _PAD))
  w1 = _pad_to(p["W1"], (_LAYERS, _PAD, _HID))
  b1 = _pad_to(p["b1"], (_LAYERS, _HID))[:, None, :]
  w2 = _pad_to(p["W2"], (_LAYERS, _HID, _PAD))
  b2 = _pad_to(p["b2"], (_LAYERS, _PAD))[:, None, :]
  g = _pad_to(p["bn_g"], (_LAYERS, _PAD))[:, None, :]
  b = _pad_to(p["bn_b"], (_LAYERS, _PAD))[:, None, :]
  return t9, combo, selfrow, w1, b1, w2, b2, g, b


def _encode(x, edge_index, edge_attr, batch, p, zeros_n, zeros_c, eye9):
  t9, combo, selfrow, w1, b1, w2, b2, g, b = _prep_encoder(p)
  cx = (x[:, 0] * 3 + x[:, 1]).astype(jnp.float32)[:, None]

  def to4d(v):
    return jnp.transpose(v.reshape(_TILES, _NCH, 1, _CHUNK), (1, 0, 2, 3))

  src2 = to4d(edge_index[0])
  dst2 = to4d(edge_index[1])
  ce2 = to4d((edge_attr[:, 0] * 3 + edge_attr[:, 1]).astype(jnp.int32))
  batch3d = batch.astype(jnp.float32).reshape(_GRID, 1, _RB)

  cmat, _ = _sc_scatter_cnt(eye9, eye9, ce2, dst2, zeros_c)

  h = _embed(cx, t9)
  for l in range(_LAYERS):
    h_a = h[:, :_HALF]
    h_b = h[:, _HALF:]
    agg_a, agg_b = _sc_scatter_h(h_a, h_b, src2, dst2, zeros_n)
    agg = jnp.concatenate([agg_a, agg_b], axis=1)
    y = _mlp(agg, h, cmat, combo[l], selfrow[l], w1[l], b1[l], w2[l], b2[l])
    h = (_bn_relu if l < _LAYERS - 1 else _bn_last)(y, g[l], b[l])
  return _pool(batch3d, h)


def kernel(x0, edge_index0, edge_attr0, batch0, x1, edge_index1, edge_attr1,
           batch1, enc0, enc1, proj):
  zeros_n = jnp.zeros((_N, _HALF), jnp.float32)
  zeros_c = jnp.zeros((_N, 16), jnp.float32)
  eye9 = jnp.eye(9, 16, dtype=jnp.float32)

  p0, c0 = _encode(x0, edge_index0, edge_attr0, batch0, enc0,
                   zeros_n, zeros_c, eye9)
  p1, c1 = _encode(x1, edge_index1, edge_attr1, batch1, enc1,
                   zeros_n, zeros_c, eye9)

  pw1 = _pad_to(proj["W1"], (_PAD, _PAD))
  pb1 = _pad_to(proj["b1"], (_PAD,))[None, :]
  pw2 = _pad_to(proj["W2"], (_PAD, _PAD))
  pb2 = _pad_to(proj["b2"], (_PAD,))[None, :]
  logits = _head(p0, c0, p1, c1, pw1, pb1, pw2, pb2)
  labels = jnp.arange(_G, dtype=jnp.int32)
  return (logits, labels)


# speedup vs baseline: 5.6311x; 1.7460x over previous
import functools

import jax
import jax.numpy as jnp
from jax import lax
from jax.experimental import pallas as pl
from jax.experimental.pallas import tpu as pltpu
from jax.experimental.pallas import tpu_sc as plsc

_EMB = 300
_PAD = 320
_HALF = 160
_HID = 640
_N = 10000
_E = 160000
_G = 512
_LAYERS = 5
_TEMP = 0.04
_TILES = 16
_CHUNK = 80
_EPT = _E // _TILES
_NCH = _EPT // _CHUNK
_ROWS_PT = 624
_RB = 1000
_GRID = _N // _RB
_PREC = jax.lax.Precision.HIGHEST


@functools.lru_cache(maxsize=None)
def _make_sc_scatter(d_half):
  mesh = plsc.VectorSubcoreMesh(core_axis_name="c", subcore_axis_name="s")
  out_t = (jax.ShapeDtypeStruct((_N, d_half), jnp.float32),
           jax.ShapeDtypeStruct((_N, d_half), jnp.float32))

  @functools.partial(
      pl.kernel,
      out_type=out_t,
      mesh=mesh,
      compiler_params=pltpu.CompilerParams(use_tc_tiling_on_sc=False),
      scratch_types=[
          pltpu.VMEM((_CHUNK,), jnp.int32),
          pltpu.VMEM((_CHUNK,), jnp.int32),
          pltpu.VMEM((_CHUNK,), jnp.int32),
          pltpu.VMEM((_CHUNK,), jnp.int32),
          pltpu.VMEM((_CHUNK, d_half), jnp.float32),
          pltpu.VMEM((_CHUNK, d_half), jnp.float32),
          pltpu.VMEM_SHARED((_N, d_half), jnp.float32),
          pltpu.SemaphoreType.DMA,
          pltpu.SemaphoreType.DMA,
      ],
  )
  def sc_kernel(tab_a, tab_b, idx4, seg4, zeros_hbm, out_a, out_b,
                idx_p, seg_p, idx_q, seg_q, rows_p, rows_q, acc_sh,
                gsem_p, gsem_q):
    c = lax.axis_index("c")
    s = lax.axis_index("s")

    def tile_rows_copy(src, dst):
      pltpu.sync_copy(src.at[pl.ds(s * _ROWS_PT, _ROWS_PT)],
                      dst.at[pl.ds(s * _ROWS_PT, _ROWS_PT)])

      @pl.when(s == _TILES - 1)
      def _():
        pltpu.sync_copy(src.at[pl.ds(_TILES * _ROWS_PT, _N - _TILES * _ROWS_PT)],
                        dst.at[pl.ds(_TILES * _ROWS_PT, _N - _TILES * _ROWS_PT)])

    tile_rows_copy(zeros_hbm, acc_sh)
    plsc.subcore_barrier()

    def run(tab):
      pltpu.sync_copy(idx4.at[0, s, 0], idx_p)
      pltpu.sync_copy(seg4.at[0, s, 0], seg_p)
      pltpu.async_copy(tab.at[idx_p], rows_p, gsem_p)

      @pl.loop(0, _NCH + 1, step=2)
      def _(i):
        @pl.when(i + 1 < _NCH)
        def _():
          pltpu.sync_copy(idx4.at[i + 1, s, 0], idx_q)
          pltpu.sync_copy(seg4.at[i + 1, s, 0], seg_q)
          pltpu.async_copy(tab.at[idx_q], rows_q, gsem_q)
        pltpu.make_async_copy(tab.at[idx_p], rows_p, gsem_p).wait()
        pltpu.sync_copy(rows_p, acc_sh.at[seg_p], add=True)

        @pl.when(i + 2 < _NCH)
        def _():
          pltpu.sync_copy(idx4.at[i + 2, s, 0], idx_p)
          pltpu.sync_copy(seg4.at[i + 2, s, 0], seg_p)
          pltpu.async_copy(tab.at[idx_p], rows_p, gsem_p)

        @pl.when(i + 1 < _NCH)
        def _():
          pltpu.make_async_copy(tab.at[idx_q], rows_q, gsem_q).wait()
          pltpu.sync_copy(rows_q, acc_sh.at[seg_q], add=True)

    @pl.when(c == 0)
    def _():
      run(tab_a)

    @pl.when(c == 1)
    def _():
      run(tab_b)

    plsc.subcore_barrier()

    @pl.when(c == 0)
    def _():
      tile_rows_copy(acc_sh, out_a)

    @pl.when(c == 1)
    def _():
      tile_rows_copy(acc_sh, out_b)

  return sc_kernel


def _sc_scatter_h(*a):
  return _make_sc_scatter(_HALF)(*a)


def _sc_scatter_cnt(*a):
  return _make_sc_scatter(16)(*a)


def _embed_body(cx_ref, tab_ref, out_ref):
  c = cx_ref[...]
  oh = (lax.broadcasted_iota(jnp.int32, (_RB, 16), 1).astype(jnp.float32) == c)
  out_ref[...] = jnp.dot(oh.astype(jnp.float32), tab_ref[...],
                         preferred_element_type=jnp.float32, precision=_PREC)


def _embed(cx, tab):
  return pl.pallas_call(
      _embed_body,
      grid=(_GRID,),
      in_specs=[
          pl.BlockSpec((_RB, 1), lambda i: (i, 0)),
          pl.BlockSpec((16, _PAD), lambda i: (0, 0)),
      ],
      out_specs=pl.BlockSpec((_RB, _PAD), lambda i: (i, 0)),
      out_shape=jax.ShapeDtypeStruct((_N, _PAD), jnp.float32),
  )(cx, tab)


def _mlp_body(agg_ref, h_ref, c_ref, combo_ref, selfrow_ref,
              w1_ref, b1_ref, w2_ref, b2_ref, y_ref):
  pre = (agg_ref[...] + h_ref[...] + selfrow_ref[...]
         + jnp.dot(c_ref[...], combo_ref[...],
                   preferred_element_type=jnp.float32, precision=_PREC))
  t = jnp.maximum(
      jnp.dot(pre, w1_ref[...], preferred_element_type=jnp.float32,
              precision=_PREC) + b1_ref[...], 0.0)
  y_ref[...] = jnp.dot(t, w2_ref[...], preferred_element_type=jnp.float32,
                       precision=_PREC) + b2_ref[...]


def _mlp(agg, h, cmat, combo, selfrow, w1, b1, w2, b2):
  return pl.pallas_call(
      _mlp_body,
      grid=(_GRID,),
      in_specs=[
          pl.BlockSpec((_RB, _PAD), lambda i: (i, 0)),
          pl.BlockSpec((_RB, _PAD), lambda i: (i, 0)),
          pl.BlockSpec((_RB, 16), lambda i: (i, 0)),
          pl.BlockSpec((16, _PAD), lambda i: (0, 0)),
          pl.BlockSpec((1, _PAD), lambda i: (0, 0)),
          pl.BlockSpec((_PAD, _HID), lambda i: (0, 0)),
          pl.BlockSpec((1, _HID), lambda i: (0, 0)),
          pl.BlockSpec((_HID, _PAD), lambda i: (0, 0)),
          pl.BlockSpec((1, _PAD), lambda i: (0, 0)),
      ],
      out_specs=pl.BlockSpec((_RB, _PAD), lambda i: (i, 0)),
      out_shape=jax.ShapeDtypeStruct((_N, _PAD), jnp.float32),
  )(agg, h, cmat, combo, selfrow, w1, b1, w2, b2)


def _make_bn(do_relu):
  def body(y_ref, g_ref, b_ref, out_ref):
    y = y_ref[...]
    mu = jnp.mean(y, axis=0, keepdims=True)
    d = y - mu
    var = jnp.mean(d * d, axis=0, keepdims=True)
    h = d / jnp.sqrt(var + 1e-5) * g_ref[...] + b_ref[...]
    if do_relu:
      h = jnp.maximum(h, 0.0)
    out_ref[...] = h

  def bn(y, g, b):
    return pl.pallas_call(
        body,
        out_shape=jax.ShapeDtypeStruct((_N, _PAD), jnp.float32),
    )(y, g, b)
  return bn


_bn_relu = _make_bn(True)
_bn_last = _make_bn(False)


def _pool_body(b_ref, h_ref, psum_ref, pcnt_ref):
  @pl.when(pl.program_id(0) == 0)
  def _():
    psum_ref[...] = jnp.zeros_like(psum_ref)
    pcnt_ref[...] = jnp.zeros_like(pcnt_ref)
  brow = b_ref[...].reshape(1, _RB)
  oh_t = (lax.broadcasted_iota(jnp.int32, (_G, _RB), 0).astype(jnp.float32)
          == brow)
  oh_t = oh_t.astype(jnp.float32)
  psum_ref[...] += jnp.dot(oh_t, h_ref[...],
                           preferred_element_type=jnp.float32,
                           precision=_PREC)
  cnt = jnp.sum(oh_t, axis=1, keepdims=True)
  pcnt_ref[...] += jnp.broadcast_to(cnt, (_G, 128))


def _pool(batch3d, h):
  return pl.pallas_call(
      _pool_body,
      grid=(_GRID,),
      in_specs=[
          pl.BlockSpec((1, 1, _RB), lambda i: (i, 0, 0)),
          pl.BlockSpec((_RB, _PAD), lambda i: (i, 0)),
      ],
      out_specs=[
          pl.BlockSpec((_G, _PAD), lambda i: (0, 0)),
          pl.BlockSpec((_G, 128), lambda i: (0, 0)),
      ],
      out_shape=[
          jax.ShapeDtypeStruct((_G, _PAD), jnp.float32),
          jax.ShapeDtypeStruct((_G, 128), jnp.float32),
      ],
  )(batch3d, h)


def _head_body(p0_ref, c0_ref, p1_ref, c1_ref, w1_ref, b1_ref, w2_ref,
               b2_ref, out_ref):
  def feat(p_ref, c_ref):
    cnt = jnp.maximum(c_ref[:, 0:1], 1.0)
    m = p_ref[...] / cnt
    t = jnp.maximum(
        jnp.dot(m, w1_ref[...], preferred_element_type=jnp.float32,
                precision=_PREC) + b1_ref[...], 0.0)
    z = jnp.dot(t, w2_ref[...], preferred_element_type=jnp.float32,
                precision=_PREC) + b2_ref[...]
    nrm = jnp.sqrt(jnp.sum(z * z, axis=1, keepdims=True))
    return z / jnp.maximum(nrm, 1e-12)
  f0 = feat(p0_ref, c0_ref)
  f1 = feat(p1_ref, c1_ref)
  out_ref[...] = lax.dot_general(
      f0, f1, (((1,), (1,)), ((), ())),
      preferred_element_type=jnp.float32, precision=_PREC) * (1.0 / _TEMP)


def _head(p0, c0, p1, c1, w1, b1, w2, b2):
  return pl.pallas_call(
      _head_body,
      out_shape=jax.ShapeDtypeStruct((_G, _G), jnp.float32),
  )(p0, c0, p1, c1, w1, b1, w2, b2)


def _pad_to(a, shape):
  pads = [(0, t - s) for s, t in zip(a.shape, shape)]
  return jnp.pad(a, pads)


def _prep_encoder(p):
  a = jnp.arange(9)
  t9 = p["atom_emb1"][a // 3] + p["atom_emb2"][a % 3]
  t9 = _pad_to(t9, (16, _PAD))
  combo = p["ee1"][:, a // 3, :] + p["ee2"][:, a % 3, :]
  combo = _pad_to(combo, (_LAYERS, 16, _PAD))
  selfrow = (p["ee1"][:, 4, :] + p["ee2"][:, 0, :])[:, None, :]
  selfrow = _pad_to(selfrow, (_LAYERS, 1, _PAD))
  w1 = _pad_to(p["W1"], (_LAYERS, _PAD, _HID))
  b1 = _pad_to(p["b1"], (_LAYERS, _HID))[:, None, :]
  w2 = _pad_to(p["W2"], (_LAYERS, _HID, _PAD))
  b2 = _pad_to(p["b2"], (_LAYERS, _PAD))[:, None, :]
  g = _pad_to(p["bn_g"], (_LAYERS, _PAD))[:, None, :]
  b = _pad_to(p["bn_b"], (_LAYERS, _PAD))[:, None, :]
  return t9, combo, selfrow, w1, b1, w2, b2, g, b


def _encode(x, edge_index, edge_attr, batch, p, zeros_n, zeros_c, eye9):
  t9, combo, selfrow, w1, b1, w2, b2, g, b = _prep_encoder(p)
  cx = (x[:, 0] * 3 + x[:, 1]).astype(jnp.float32)[:, None]

  def to4d(v):
    return jnp.transpose(v.reshape(_TILES, _NCH, 1, _CHUNK), (1, 0, 2, 3))

  src2 = to4d(edge_index[0])
  dst2 = to4d(edge_index[1])
  ce = (edge_attr[:, 0] * 3 + edge_attr[:, 1]).astype(jnp.int32)
  ce = ce.reshape(_TILES, _EPT) + 9 * jnp.arange(_TILES, dtype=jnp.int32)[:, None]
  ce2 = to4d(ce)
  batch3d = batch.astype(jnp.float32).reshape(_GRID, 1, _RB)

  cmat, _ = _sc_scatter_cnt(eye9, eye9, ce2, dst2, zeros_c)

  h = _embed(cx, t9)
  for l in range(_LAYERS):
    h_a = h[:, :_HALF]
    h_b = h[:, _HALF:]
    agg_a, agg_b = _sc_scatter_h(h_a, h_b, src2, dst2, zeros_n)
    agg = jnp.concatenate([agg_a, agg_b], axis=1)
    y = _mlp(agg, h, cmat, combo[l], selfrow[l], w1[l], b1[l], w2[l], b2[l])
    h = (_bn_relu if l < _LAYERS - 1 else _bn_last)(y, g[l], b[l])
  return _pool(batch3d, h)


def kernel(x0, edge_index0, edge_attr0, batch0, x1, edge_index1, edge_attr1,
           batch1, enc0, enc1, proj):
  zeros_n = jnp.zeros((_N, _HALF), jnp.float32)
  zeros_c = jnp.zeros((_N, 16), jnp.float32)
  eye9 = jnp.tile(jnp.eye(9, 16, dtype=jnp.float32), (_TILES, 1))

  p0, c0 = _encode(x0, edge_index0, edge_attr0, batch0, enc0,
                   zeros_n, zeros_c, eye9)
  p1, c1 = _encode(x1, edge_index1, edge_attr1, batch1, enc1,
                   zeros_n, zeros_c, eye9)

  pw1 = _pad_to(proj["W1"], (_PAD, _PAD))
  pb1 = _pad_to(proj["b1"], (_PAD,))[None, :]
  pw2 = _pad_to(proj["W2"], (_PAD, _PAD))
  pb2 = _pad_to(proj["b2"], (_PAD,))[None, :]
  logits = _head(p0, c0, p1, c1, pw1, pb1, pw2, pb2)
  labels = jnp.arange(_G, dtype=jnp.int32)
  return (logits, labels)


# speedup vs baseline: 7.1219x; 1.2648x over previous
import functools

import jax
import jax.numpy as jnp
from jax import lax
from jax.experimental import pallas as pl
from jax.experimental.pallas import tpu as pltpu
from jax.experimental.pallas import tpu_sc as plsc

_EMB = 300
_PAD = 320
_HALF = 160
_HID = 640
_N = 10000
_E = 160000
_G = 512
_LAYERS = 5
_TEMP = 0.04
_TILES = 16
_CHUNK = 80
_EPT = _E // _TILES
_NCH = _EPT // _CHUNK
_SCH = 25
_NST = _NCH // _SCH
_ROWS_PT = 624
_RB = 1000
_GRID = _N // _RB
_PREC = jax.lax.Precision.HIGHEST


@functools.lru_cache(maxsize=None)
def _make_sc_scatter(d_half):
  mesh = plsc.VectorSubcoreMesh(core_axis_name="c", subcore_axis_name="s")
  out_t = (jax.ShapeDtypeStruct((_N, d_half), jnp.float32),
           jax.ShapeDtypeStruct((_N, d_half), jnp.float32))

  @functools.partial(
      pl.kernel,
      out_type=out_t,
      mesh=mesh,
      compiler_params=pltpu.CompilerParams(use_tc_tiling_on_sc=False),
      scratch_types=[
          pltpu.VMEM((_SCH, _CHUNK), jnp.int32),
          pltpu.VMEM((_SCH, _CHUNK), jnp.int32),
          pltpu.VMEM((_CHUNK, d_half), jnp.float32),
          pltpu.VMEM((_CHUNK, d_half), jnp.float32),
          pltpu.VMEM_SHARED((_N, d_half), jnp.float32),
          pltpu.SemaphoreType.DMA,
          pltpu.SemaphoreType.DMA,
      ],
  )
  def sc_kernel(tab_a, tab_b, idx4, seg4, zeros_hbm, out_a, out_b,
                idx_st, seg_st, rows_p, rows_q, acc_sh,
                gsem_p, gsem_q):
    c = lax.axis_index("c")
    s = lax.axis_index("s")

    def tile_rows_copy(src, dst):
      pltpu.sync_copy(src.at[pl.ds(s * _ROWS_PT, _ROWS_PT)],
                      dst.at[pl.ds(s * _ROWS_PT, _ROWS_PT)])

      @pl.when(s == _TILES - 1)
      def _():
        pltpu.sync_copy(src.at[pl.ds(_TILES * _ROWS_PT, _N - _TILES * _ROWS_PT)],
                        dst.at[pl.ds(_TILES * _ROWS_PT, _N - _TILES * _ROWS_PT)])

    tile_rows_copy(zeros_hbm, acc_sh)
    plsc.subcore_barrier()

    def run(tab):
      def stage(j, carry):
        pltpu.sync_copy(idx4.at[j, s], idx_st)
        pltpu.sync_copy(seg4.at[j, s], seg_st)
        pltpu.async_copy(tab.at[idx_st.at[0]], rows_p, gsem_p)

        @pl.loop(0, _SCH + 1, step=2)
        def _(i):
          @pl.when(i + 1 < _SCH)
          def _():
            pltpu.async_copy(tab.at[idx_st.at[i + 1]], rows_q, gsem_q)
          pltpu.make_async_copy(tab.at[idx_st.at[i]], rows_p, gsem_p).wait()
          pltpu.sync_copy(rows_p, acc_sh.at[seg_st.at[i]], add=True)

          @pl.when(i + 2 < _SCH)
          def _():
            pltpu.async_copy(tab.at[idx_st.at[i + 2]], rows_p, gsem_p)

          @pl.when(i + 1 < _SCH)
          def _():
            pltpu.make_async_copy(tab.at[idx_st.at[i + 1]], rows_q, gsem_q).wait()
            pltpu.sync_copy(rows_q, acc_sh.at[seg_st.at[i + 1]], add=True)
        return carry
      lax.fori_loop(0, _NST, stage, 0)

    @pl.when(c == 0)
    def _():
      run(tab_a)

    @pl.when(c == 1)
    def _():
      run(tab_b)

    plsc.subcore_barrier()

    @pl.when(c == 0)
    def _():
      tile_rows_copy(acc_sh, out_a)

    @pl.when(c == 1)
    def _():
      tile_rows_copy(acc_sh, out_b)

  return sc_kernel


def _sc_scatter_h(*a):
  return _make_sc_scatter(_HALF)(*a)


def _sc_scatter_cnt(*a):
  return _make_sc_scatter(16)(*a)


def _embed_body(cx_ref, tab_ref, out_ref):
  c = cx_ref[...]
  oh = (lax.broadcasted_iota(jnp.int32, (_RB, 16), 1).astype(jnp.float32) == c)
  out_ref[...] = jnp.dot(oh.astype(jnp.float32), tab_ref[...],
                         preferred_element_type=jnp.float32, precision=_PREC)


def _embed(cx, tab):
  return pl.pallas_call(
      _embed_body,
      grid=(_GRID,),
      in_specs=[
          pl.BlockSpec((_RB, 1), lambda i: (i, 0)),
          pl.BlockSpec((16, _PAD), lambda i: (0, 0)),
      ],
      out_specs=pl.BlockSpec((_RB, _PAD), lambda i: (i, 0)),
      out_shape=jax.ShapeDtypeStruct((_N, _PAD), jnp.float32),
  )(cx, tab)


def _mlp_body(agg_ref, h_ref, c_ref, combo_ref, selfrow_ref,
              w1_ref, b1_ref, w2_ref, b2_ref, y_ref):
  pre = (agg_ref[...] + h_ref[...] + selfrow_ref[...]
         + jnp.dot(c_ref[...], combo_ref[...],
                   preferred_element_type=jnp.float32, precision=_PREC))
  t = jnp.maximum(
      jnp.dot(pre, w1_ref[...], preferred_element_type=jnp.float32,
              precision=_PREC) + b1_ref[...], 0.0)
  y_ref[...] = jnp.dot(t, w2_ref[...], preferred_element_type=jnp.float32,
                       precision=_PREC) + b2_ref[...]


def _mlp(agg, h, cmat, combo, selfrow, w1, b1, w2, b2):
  return pl.pallas_call(
      _mlp_body,
      grid=(_GRID,),
      in_specs=[
          pl.BlockSpec((_RB, _PAD), lambda i: (i, 0)),
          pl.BlockSpec((_RB, _PAD), lambda i: (i, 0)),
          pl.BlockSpec((_RB, 16), lambda i: (i, 0)),
          pl.BlockSpec((16, _PAD), lambda i: (0, 0)),
          pl.BlockSpec((1, _PAD), lambda i: (0, 0)),
          pl.BlockSpec((_PAD, _HID), lambda i: (0, 0)),
          pl.BlockSpec((1, _HID), lambda i: (0, 0)),
          pl.BlockSpec((_HID, _PAD), lambda i: (0, 0)),
          pl.BlockSpec((1, _PAD), lambda i: (0, 0)),
      ],
      out_specs=pl.BlockSpec((_RB, _PAD), lambda i: (i, 0)),
      out_shape=jax.ShapeDtypeStruct((_N, _PAD), jnp.float32),
  )(agg, h, cmat, combo, selfrow, w1, b1, w2, b2)


def _make_bn(do_relu):
  def body(y_ref, g_ref, b_ref, out_ref):
    y = y_ref[...]
    mu = jnp.mean(y, axis=0, keepdims=True)
    d = y - mu
    var = jnp.mean(d * d, axis=0, keepdims=True)
    h = d / jnp.sqrt(var + 1e-5) * g_ref[...] + b_ref[...]
    if do_relu:
      h = jnp.maximum(h, 0.0)
    out_ref[...] = h

  def bn(y, g, b):
    return pl.pallas_call(
        body,
        out_shape=jax.ShapeDtypeStruct((_N, _PAD), jnp.float32),
    )(y, g, b)
  return bn


_bn_relu = _make_bn(True)
_bn_last = _make_bn(False)


def _pool_body(b_ref, h_ref, psum_ref, pcnt_ref):
  @pl.when(pl.program_id(0) == 0)
  def _():
    psum_ref[...] = jnp.zeros_like(psum_ref)
    pcnt_ref[...] = jnp.zeros_like(pcnt_ref)
  brow = b_ref[...].reshape(1, _RB)
  oh_t = (lax.broadcasted_iota(jnp.int32, (_G, _RB), 0).astype(jnp.float32)
          == brow)
  oh_t = oh_t.astype(jnp.float32)
  psum_ref[...] += jnp.dot(oh_t, h_ref[...],
                           preferred_element_type=jnp.float32,
                           precision=_PREC)
  cnt = jnp.sum(oh_t, axis=1, keepdims=True)
  pcnt_ref[...] += jnp.broadcast_to(cnt, (_G, 128))


def _pool(batch3d, h):
  return pl.pallas_call(
      _pool_body,
      grid=(_GRID,),
      in_specs=[
          pl.BlockSpec((1, 1, _RB), lambda i: (i, 0, 0)),
          pl.BlockSpec((_RB, _PAD), lambda i: (i, 0)),
      ],
      out_specs=[
          pl.BlockSpec((_G, _PAD), lambda i: (0, 0)),
          pl.BlockSpec((_G, 128), lambda i: (0, 0)),
      ],
      out_shape=[
          jax.ShapeDtypeStruct((_G, _PAD), jnp.float32),
          jax.ShapeDtypeStruct((_G, 128), jnp.float32),
      ],
  )(batch3d, h)


def _head_body(p0_ref, c0_ref, p1_ref, c1_ref, w1_ref, b1_ref, w2_ref,
               b2_ref, out_ref):
  def feat(p_ref, c_ref):
    cnt = jnp.maximum(c_ref[:, 0:1], 1.0)
    m = p_ref[...] / cnt
    t = jnp.maximum(
        jnp.dot(m, w1_ref[...], preferred_element_type=jnp.float32,
                precision=_PREC) + b1_ref[...], 0.0)
    z = jnp.dot(t, w2_ref[...], preferred_element_type=jnp.float32,
                precision=_PREC) + b2_ref[...]
    nrm = jnp.sqrt(jnp.sum(z * z, axis=1, keepdims=True))
    return z / jnp.maximum(nrm, 1e-12)
  f0 = feat(p0_ref, c0_ref)
  f1 = feat(p1_ref, c1_ref)
  out_ref[...] = lax.dot_general(
      f0, f1, (((1,), (1,)), ((), ())),
      preferred_element_type=jnp.float32, precision=_PREC) * (1.0 / _TEMP)


def _head(p0, c0, p1, c1, w1, b1, w2, b2):
  return pl.pallas_call(
      _head_body,
      out_shape=jax.ShapeDtypeStruct((_G, _G), jnp.float32),
  )(p0, c0, p1, c1, w1, b1, w2, b2)


def _pad_to(a, shape):
  pads = [(0, t - s) for s, t in zip(a.shape, shape)]
  return jnp.pad(a, pads)


def _prep_encoder(p):
  a = jnp.arange(9)
  t9 = p["atom_emb1"][a // 3] + p["atom_emb2"][a % 3]
  t9 = _pad_to(t9, (16, _PAD))
  combo = p["ee1"][:, a // 3, :] + p["ee2"][:, a % 3, :]
  combo = _pad_to(combo, (_LAYERS, 16, _PAD))
  selfrow = (p["ee1"][:, 4, :] + p["ee2"][:, 0, :])[:, None, :]
  selfrow = _pad_to(selfrow, (_LAYERS, 1, _PAD))
  w1 = _pad_to(p["W1"], (_LAYERS, _PAD, _HID))
  b1 = _pad_to(p["b1"], (_LAYERS, _HID))[:, None, :]
  w2 = _pad_to(p["W2"], (_LAYERS, _HID, _PAD))
  b2 = _pad_to(p["b2"], (_LAYERS, _PAD))[:, None, :]
  g = _pad_to(p["bn_g"], (_LAYERS, _PAD))[:, None, :]
  b = _pad_to(p["bn_b"], (_LAYERS, _PAD))[:, None, :]
  return t9, combo, selfrow, w1, b1, w2, b2, g, b


def _encode(x, edge_index, edge_attr, batch, p, zeros_n, zeros_c, eye9):
  t9, combo, selfrow, w1, b1, w2, b2, g, b = _prep_encoder(p)
  cx = (x[:, 0] * 3 + x[:, 1]).astype(jnp.float32)[:, None]

  def to4d(v):
    return jnp.transpose(v.reshape(_TILES, _NST, _SCH, _CHUNK), (1, 0, 2, 3))

  src2 = to4d(edge_index[0])
  dst2 = to4d(edge_index[1])
  ce = (edge_attr[:, 0] * 3 + edge_attr[:, 1]).astype(jnp.int32)
  ce = ce.reshape(_TILES, _EPT) + 9 * jnp.arange(_TILES, dtype=jnp.int32)[:, None]
  ce2 = to4d(ce.reshape(-1))
  batch3d = batch.astype(jnp.float32).reshape(_GRID, 1, _RB)

  cmat, _ = _sc_scatter_cnt(eye9, eye9, ce2, dst2, zeros_c)

  h = _embed(cx, t9)
  for l in range(_LAYERS):
    h_a = h[:, :_HALF]
    h_b = h[:, _HALF:]
    agg_a, agg_b = _sc_scatter_h(h_a, h_b, src2, dst2, zeros_n)
    agg = jnp.concatenate([agg_a, agg_b], axis=1)
    y = _mlp(agg, h, cmat, combo[l], selfrow[l], w1[l], b1[l], w2[l], b2[l])
    h = (_bn_relu if l < _LAYERS - 1 else _bn_last)(y, g[l], b[l])
  return _pool(batch3d, h)


def kernel(x0, edge_index0, edge_attr0, batch0, x1, edge_index1, edge_attr1,
           batch1, enc0, enc1, proj):
  zeros_n = jnp.zeros((_N, _HALF), jnp.float32)
  zeros_c = jnp.zeros((_N, 16), jnp.float32)
  eye9 = jnp.tile(jnp.eye(9, 16, dtype=jnp.float32), (_TILES, 1))

  p0, c0 = _encode(x0, edge_index0, edge_attr0, batch0, enc0,
                   zeros_n, zeros_c, eye9)
  p1, c1 = _encode(x1, edge_index1, edge_attr1, batch1, enc1,
                   zeros_n, zeros_c, eye9)

  pw1 = _pad_to(proj["W1"], (_PAD, _PAD))
  pb1 = _pad_to(proj["b1"], (_PAD,))[None, :]
  pw2 = _pad_to(proj["W2"], (_PAD, _PAD))
  pb2 = _pad_to(proj["b2"], (_PAD,))[None, :]
  logits = _head(p0, c0, p1, c1, pw1, pb1, pw2, pb2)
  labels = jnp.arange(_G, dtype=jnp.int32)
  return (logits, labels)
